# baseline (speedup 1.0000x reference)
import functools

import jax
import jax.numpy as jnp
import numpy as np
from jax import lax
from jax.experimental import pallas as pl
from jax.experimental.pallas import tpu as pltpu
from jax.experimental.pallas import tpu_sc as plsc

N_STATES = 128
BATCH = 1024
ROW_D = 256
ALPHA = np.float32(0.5)
UNIF_C = np.float32((1.0 - 0.5) / 128)
TINY = np.float32(np.finfo(np.float32).tiny)

_KS0 = np.uint32(0)
_KS1 = np.uint32(42)
_KS2 = np.uint32(0 ^ 42 ^ 0x1BD11BDA)
_ROTS = ((13, 15, 26, 6), (17, 29, 16, 24))


def _rotl(x, r):
    return (x << np.uint32(r)) | (x >> np.uint32(32 - r))


def _threefry_bits(x1):
    ks = (_KS0, _KS1, _KS2)
    x0 = jnp.zeros_like(x1) + ks[0]
    x1 = x1 + ks[1]
    for i in range(5):
        for r in _ROTS[i % 2]:
            x0 = x0 + x1
            x1 = _rotl(x1, r)
            x1 = x0 ^ x1
        x0 = x0 + ks[(i + 1) % 3]
        x1 = x1 + ks[(i + 2) % 3] + np.uint32(i + 1)
    return x0 ^ x1


_BC = 128


_SB = 16


def _sample_body(p_ref, idx_ref, pn_ref, ir_ref, dt_ref, pnacc_ref):
    sblk = pl.program_id(0)

    @pl.when(sblk == 0)
    def _():
        p = p_ref[...]
        rp = ALPHA * jnp.exp(p) + UNIF_C
        ir_ref[...] = np.float32(1.0) / rp
        dt_ref[...] = p - jnp.log(rp)

    iota_k = lax.broadcasted_iota(jnp.int32, (N_STATES, _BC), 0)
    iota_b = lax.broadcasted_iota(jnp.int32, (N_STATES, _BC), 1)
    flat_rows = []
    pn_rows = []
    for si in range(_SB):
        s = sblk * _SB + si
        flat_pieces = []
        pn_pieces = []
        for bc in range(BATCH // _BC):
            ir = ir_ref[:, pl.ds(bc * _BC, _BC)]
            dt = dt_ref[:, pl.ds(bc * _BC, _BC)]
            j = (
                s * (BATCH * N_STATES) + (bc * _BC + iota_b) * N_STATES + iota_k
            ).astype(jnp.uint32)
            bits = _threefry_bits(j)
            fb = (bits >> np.uint32(9)) | np.uint32(0x3F800000)
            f = lax.bitcast_convert_type(fb, jnp.float32) - np.float32(1.0)
            u = jnp.maximum(TINY, f * (np.float32(1.0) - TINY) + TINY)
            score = -jnp.log(u) * ir
            mn = jnp.min(score, axis=0, keepdims=True)
            idx = jnp.min(
                jnp.where(score == mn, iota_k, np.int32(N_STATES)),
                axis=0,
                keepdims=True,
            )
            pn = jnp.sum(
                jnp.where(iota_k == idx, dt, np.float32(0.0)), axis=0, keepdims=True
            )
            flat_pieces.append(idx * BATCH + (bc * _BC + iota_b[0:1, :]))
            pn_pieces.append(pn)
        flat_rows.append(jnp.concatenate(flat_pieces, axis=1))
        pn_rows.append(jnp.concatenate(pn_pieces, axis=1))
    idx_ref[...] = jnp.concatenate(flat_rows, axis=0)
    pnacc_ref[pl.ds(sblk * _SB, _SB), :] = jnp.concatenate(pn_rows, axis=0)

    @pl.when(sblk == (N_STATES // _SB) - 1)
    def _():
        x = pnacc_ref[...]
        m = jnp.max(x, axis=0, keepdims=True)
        lse = jnp.log(jnp.sum(jnp.exp(x - m), axis=0, keepdims=True)) + m
        pn_ref[...] = x - lse


def _sample_call(p2d):
    return pl.pallas_call(
        _sample_body,
        grid=(N_STATES // _SB,),
        in_specs=[pl.BlockSpec((N_STATES, BATCH), lambda s: (0, 0))],
        out_specs=[
            pl.BlockSpec((_SB, BATCH), lambda s: (s, 0)),
            pl.BlockSpec((N_STATES, BATCH), lambda s: (0, 0)),
        ],
        out_shape=[
            jax.ShapeDtypeStruct((N_STATES, BATCH), jnp.int32),
            jax.ShapeDtypeStruct((N_STATES, BATCH), jnp.float32),
        ],
        scratch_shapes=[
            pltpu.VMEM((N_STATES, BATCH), jnp.float32),
            pltpu.VMEM((N_STATES, BATCH), jnp.float32),
            pltpu.VMEM((N_STATES, BATCH), jnp.float32),
        ],
    )(p2d)


_N_WORKERS = 32
_CHUNK = 128


def _gather_body(
    chunks_per_w, states_hbm, idx_hbm, out_hbm, idx_v, buf0, buf1, gs0, gs1, ws0, ws1
):
    info = plsc.get_sparse_core_info()
    nc = info.num_cores
    wid = lax.axis_index("s") * nc + lax.axis_index("c")
    crow0 = wid * chunks_per_w
    pltpu.sync_copy(idx_hbm.at[pl.ds(crow0, chunks_per_w)], idx_v)

    bufs = (buf0, buf1)
    gs = (gs0, gs1)
    ws = (ws0, ws1)
    pltpu.async_copy(states_hbm.at[idx_v.at[0]], buf0, gs0)
    pltpu.async_copy(states_hbm.at[idx_v.at[1]], buf1, gs1)

    @pl.loop(0, chunks_per_w, step=2)
    def _(c0):
        for b in range(2):
            c = c0 + b
            pltpu.make_async_copy(states_hbm.at[idx_v.at[c]], bufs[b], gs[b]).wait()
            row0 = (crow0 + c) * _CHUNK
            dst = out_hbm.at[pl.ds(row0, _CHUNK)]
            pltpu.async_copy(bufs[b], dst, ws[b])

            @pl.when(c + 2 < chunks_per_w)
            def _():
                pltpu.make_async_copy(bufs[b], dst, ws[b]).wait()
                pltpu.async_copy(states_hbm.at[idx_v.at[c + 2]], bufs[b], gs[b])

    for b in range(2):
        c_last = chunks_per_w - 2 + b
        row0 = (crow0 + c_last) * _CHUNK
        pltpu.make_async_copy(bufs[b], out_hbm.at[pl.ds(row0, _CHUNK)], ws[b]).wait()


def _gather_call(states, idx2d):
    nrows = idx2d.shape[0] * idx2d.shape[1]
    chunks_per_w = nrows // (_N_WORKERS * _CHUNK)
    mesh = plsc.VectorSubcoreMesh(core_axis_name="c", subcore_axis_name="s")
    f = pl.kernel(
        functools.partial(_gather_body, chunks_per_w),
        out_type=jax.ShapeDtypeStruct((nrows, ROW_D), jnp.float32),
        mesh=mesh,
        scratch_types=[
            pltpu.VMEM((chunks_per_w, _CHUNK), jnp.int32),
            pltpu.VMEM((_CHUNK, ROW_D), jnp.float32),
            pltpu.VMEM((_CHUNK, ROW_D), jnp.float32),
            pltpu.SemaphoreType.DMA,
            pltpu.SemaphoreType.DMA,
            pltpu.SemaphoreType.DMA,
            pltpu.SemaphoreType.DMA,
        ],
    )
    return f(states, idx2d)


def kernel(states, prob):
    p2d = prob.reshape(N_STATES, BATCH)
    flat3, pn2d = _sample_call(p2d)
    prob_new = pn2d.reshape(-1, 1)
    new_states = _gather_call(states, flat3.reshape(-1, _CHUNK))
    return new_states, prob_new


# speedup vs baseline: 1.2305x; 1.2305x over previous
import functools

import jax
import jax.numpy as jnp
import numpy as np
from jax import lax
from jax.experimental import pallas as pl
from jax.experimental.pallas import tpu as pltpu
from jax.experimental.pallas import tpu_sc as plsc

N_STATES = 128
BATCH = 1024
ROW_D = 256
ALPHA = np.float32(0.5)
UNIF_C = np.float32((1.0 - 0.5) / 128)
TINY = np.float32(np.finfo(np.float32).tiny)

_KS0 = np.uint32(0)
_KS1 = np.uint32(42)
_KS2 = np.uint32(0 ^ 42 ^ 0x1BD11BDA)
_ROTS = ((13, 15, 26, 6), (17, 29, 16, 24))


def _rotl(x, r):
    return (x << np.uint32(r)) | (x >> np.uint32(32 - r))


def _threefry_bits(x1):
    ks = (_KS0, _KS1, _KS2)
    x0 = jnp.zeros_like(x1) + ks[0]
    x1 = x1 + ks[1]
    for i in range(5):
        for r in _ROTS[i % 2]:
            x0 = x0 + x1
            x1 = _rotl(x1, r)
            x1 = x0 ^ x1
        x0 = x0 + ks[(i + 1) % 3]
        x1 = x1 + ks[(i + 2) % 3] + np.uint32(i + 1)
    return x0 ^ x1


_BC = 128


_SB = 8


def _sample_body(p_ref, idx_ref, pn_ref, ir_ref, dt_ref, pnacc_ref):
    sblk = pl.program_id(0)

    @pl.when(sblk == 0)
    def _():
        p = p_ref[...]
        rp = ALPHA * jnp.exp(p) + UNIF_C
        ir_ref[...] = np.float32(1.0) / rp
        dt_ref[...] = p - jnp.log(rp)

    iota_k = lax.broadcasted_iota(jnp.int32, (N_STATES, _BC), 0)
    iota_b = lax.broadcasted_iota(jnp.int32, (N_STATES, _BC), 1)
    flat_rows = []
    pn_rows = []
    for si in range(_SB):
        s = sblk * _SB + si
        flat_pieces = []
        pn_pieces = []
        for bc in range(BATCH // _BC):
            ir = ir_ref[:, pl.ds(bc * _BC, _BC)]
            dt = dt_ref[:, pl.ds(bc * _BC, _BC)]
            j = (
                s * (BATCH * N_STATES) + (bc * _BC + iota_b) * N_STATES + iota_k
            ).astype(jnp.uint32)
            bits = _threefry_bits(j)
            fb = (bits >> np.uint32(9)) | np.uint32(0x3F800000)
            f = lax.bitcast_convert_type(fb, jnp.float32) - np.float32(1.0)
            u = jnp.maximum(TINY, f * (np.float32(1.0) - TINY) + TINY)
            score = -jnp.log(u) * ir
            mn = jnp.min(score, axis=0, keepdims=True)
            idx = jnp.min(
                jnp.where(score == mn, iota_k, np.int32(N_STATES)),
                axis=0,
                keepdims=True,
            )
            pn = jnp.sum(
                jnp.where(iota_k == idx, dt, np.float32(0.0)), axis=0, keepdims=True
            )
            flat_pieces.append(idx * BATCH + (bc * _BC + iota_b[0:1, :]))
            pn_pieces.append(pn)
        flat_rows.append(jnp.concatenate(flat_pieces, axis=1))
        pn_rows.append(jnp.concatenate(pn_pieces, axis=1))
    idx_ref[...] = jnp.concatenate(flat_rows, axis=0)
    pnacc_ref[pl.ds(sblk * _SB, _SB), :] = jnp.concatenate(pn_rows, axis=0)

    @pl.when(sblk == (N_STATES // _SB) - 1)
    def _():
        x = pnacc_ref[...]
        m = jnp.max(x, axis=0, keepdims=True)
        lse = jnp.log(jnp.sum(jnp.exp(x - m), axis=0, keepdims=True)) + m
        pn_ref[...] = x - lse


def _sample_call(p2d):
    return pl.pallas_call(
        _sample_body,
        grid=(N_STATES // _SB,),
        in_specs=[pl.BlockSpec((N_STATES, BATCH), lambda s: (0, 0))],
        out_specs=[
            pl.BlockSpec((_SB, BATCH), lambda s: (s, 0)),
            pl.BlockSpec((N_STATES, BATCH), lambda s: (0, 0)),
        ],
        out_shape=[
            jax.ShapeDtypeStruct((N_STATES, BATCH), jnp.int32),
            jax.ShapeDtypeStruct((N_STATES, BATCH), jnp.float32),
        ],
        scratch_shapes=[
            pltpu.VMEM((N_STATES, BATCH), jnp.float32),
            pltpu.VMEM((N_STATES, BATCH), jnp.float32),
            pltpu.VMEM((N_STATES, BATCH), jnp.float32),
        ],
    )(p2d)


_N_WORKERS = 32
_CHUNK = 128


def _gather_body(
    chunks_per_w, states_hbm, idx_hbm, out_hbm, idx_v, buf0, buf1, gs0, gs1, ws0, ws1
):
    info = plsc.get_sparse_core_info()
    nc = info.num_cores
    wid = lax.axis_index("s") * nc + lax.axis_index("c")
    crow0 = wid * chunks_per_w
    pltpu.sync_copy(idx_hbm.at[pl.ds(crow0, chunks_per_w)], idx_v)

    bufs = (buf0, buf1)
    gs = (gs0, gs1)
    ws = (ws0, ws1)
    pltpu.async_copy(states_hbm.at[idx_v.at[0]], buf0, gs0)
    pltpu.async_copy(states_hbm.at[idx_v.at[1]], buf1, gs1)

    @pl.loop(0, chunks_per_w, step=2)
    def _(c0):
        for b in range(2):
            c = c0 + b
            pltpu.make_async_copy(states_hbm.at[idx_v.at[c]], bufs[b], gs[b]).wait()
            row0 = (crow0 + c) * _CHUNK
            dst = out_hbm.at[pl.ds(row0, _CHUNK)]
            pltpu.async_copy(bufs[b], dst, ws[b])

            @pl.when(c + 2 < chunks_per_w)
            def _():
                pltpu.make_async_copy(bufs[b], dst, ws[b]).wait()
                pltpu.async_copy(states_hbm.at[idx_v.at[c + 2]], bufs[b], gs[b])

    for b in range(2):
        c_last = chunks_per_w - 2 + b
        row0 = (crow0 + c_last) * _CHUNK
        pltpu.make_async_copy(bufs[b], out_hbm.at[pl.ds(row0, _CHUNK)], ws[b]).wait()


def _gather_call(states, idx2d):
    nrows = idx2d.shape[0] * idx2d.shape[1]
    chunks_per_w = nrows // (_N_WORKERS * _CHUNK)
    mesh = plsc.VectorSubcoreMesh(core_axis_name="c", subcore_axis_name="s")
    f = pl.kernel(
        functools.partial(_gather_body, chunks_per_w),
        out_type=jax.ShapeDtypeStruct((nrows, ROW_D), jnp.float32),
        mesh=mesh,
        scratch_types=[
            pltpu.VMEM((chunks_per_w, _CHUNK), jnp.int32),
            pltpu.VMEM((_CHUNK, ROW_D), jnp.float32),
            pltpu.VMEM((_CHUNK, ROW_D), jnp.float32),
            pltpu.SemaphoreType.DMA,
            pltpu.SemaphoreType.DMA,
            pltpu.SemaphoreType.DMA,
            pltpu.SemaphoreType.DMA,
        ],
    )
    return f(states, idx2d)


def kernel(states, prob):
    p2d = prob.reshape(N_STATES, BATCH)
    flat3, pn2d = _sample_call(p2d)
    prob_new = pn2d.reshape(-1, 1)
    new_states = _gather_call(states, flat3.reshape(-1, _CHUNK))
    return new_states, prob_new


# speedup vs baseline: 1.2320x; 1.0012x over previous
import functools

import jax
import jax.numpy as jnp
import numpy as np
from jax import lax
from jax.experimental import pallas as pl
from jax.experimental.pallas import tpu as pltpu
from jax.experimental.pallas import tpu_sc as plsc

N_STATES = 128
BATCH = 1024
ROW_D = 256
ALPHA = np.float32(0.5)
UNIF_C = np.float32((1.0 - 0.5) / 128)
TINY = np.float32(np.finfo(np.float32).tiny)

_KS0 = np.uint32(0)
_KS1 = np.uint32(42)
_KS2 = np.uint32(0 ^ 42 ^ 0x1BD11BDA)
_ROTS = ((13, 15, 26, 6), (17, 29, 16, 24))


def _rotl(x, r):
    return (x << np.uint32(r)) | (x >> np.uint32(32 - r))


def _threefry_bits(x1):
    ks = (_KS0, _KS1, _KS2)
    x0 = jnp.zeros_like(x1) + ks[0]
    x1 = x1 + ks[1]
    for i in range(5):
        for r in _ROTS[i % 2]:
            x0 = x0 + x1
            x1 = _rotl(x1, r)
            x1 = x0 ^ x1
        x0 = x0 + ks[(i + 1) % 3]
        x1 = x1 + ks[(i + 2) % 3] + np.uint32(i + 1)
    return x0 ^ x1


_BC = 128


_SB = 8


def _sample_body(p_ref, idx_ref, pn_ref, ir_ref, dt_ref, pnacc_ref):
    sblk = pl.program_id(0)

    @pl.when(sblk == 0)
    def _():
        p = p_ref[...]
        rp = ALPHA * jnp.exp(p) + UNIF_C
        ir_ref[...] = np.float32(1.0) / rp
        dt_ref[...] = p - jnp.log(rp)

    iota_k = lax.broadcasted_iota(jnp.int32, (N_STATES, _BC), 0)
    iota_b = lax.broadcasted_iota(jnp.int32, (N_STATES, _BC), 1)
    flat_rows = []
    pn_rows = []
    for si in range(_SB):
        s = sblk * _SB + si
        flat_pieces = []
        pn_pieces = []
        for bc in range(BATCH // _BC):
            ir = ir_ref[:, pl.ds(bc * _BC, _BC)]
            dt = dt_ref[:, pl.ds(bc * _BC, _BC)]
            j = (
                s * (BATCH * N_STATES) + (bc * _BC + iota_b) * N_STATES + iota_k
            ).astype(jnp.uint32)
            bits = _threefry_bits(j)
            fb = (bits >> np.uint32(9)) | np.uint32(0x3F800000)
            f = lax.bitcast_convert_type(fb, jnp.float32) - np.float32(1.0)
            u = jnp.maximum(TINY, f * (np.float32(1.0) - TINY) + TINY)
            score = -jnp.log(u) * ir
            mn = jnp.min(score, axis=0, keepdims=True)
            idx = jnp.min(
                jnp.where(score == mn, iota_k, np.int32(N_STATES)),
                axis=0,
                keepdims=True,
            )
            pn = jnp.sum(
                jnp.where(iota_k == idx, dt, np.float32(0.0)), axis=0, keepdims=True
            )
            flat_pieces.append(idx * BATCH + (bc * _BC + iota_b[0:1, :]))
            pn_pieces.append(pn)
        flat_rows.append(jnp.concatenate(flat_pieces, axis=1))
        pn_rows.append(jnp.concatenate(pn_pieces, axis=1))
    idx_ref[...] = jnp.concatenate(flat_rows, axis=0)
    pnacc_ref[pl.ds(sblk * _SB, _SB), :] = jnp.concatenate(pn_rows, axis=0)

    @pl.when(sblk == (N_STATES // _SB) - 1)
    def _():
        x = pnacc_ref[...]
        m = jnp.max(x, axis=0, keepdims=True)
        lse = jnp.log(jnp.sum(jnp.exp(x - m), axis=0, keepdims=True)) + m
        pn_ref[...] = x - lse


def _sample_call(p2d):
    return pl.pallas_call(
        _sample_body,
        grid=(N_STATES // _SB,),
        in_specs=[pl.BlockSpec((N_STATES, BATCH), lambda s: (0, 0))],
        out_specs=[
            pl.BlockSpec((_SB, BATCH), lambda s: (s, 0)),
            pl.BlockSpec((N_STATES, BATCH), lambda s: (0, 0)),
        ],
        out_shape=[
            jax.ShapeDtypeStruct((N_STATES, BATCH), jnp.int32),
            jax.ShapeDtypeStruct((N_STATES, BATCH), jnp.float32),
        ],
        scratch_shapes=[
            pltpu.VMEM((N_STATES, BATCH), jnp.float32),
            pltpu.VMEM((N_STATES, BATCH), jnp.float32),
            pltpu.VMEM((N_STATES, BATCH), jnp.float32),
        ],
    )(p2d)


_N_WORKERS = 32
_CHUNK = 128


def _gather_body(
    chunks_per_w,
    states_hbm,
    idx_hbm,
    out_hbm,
    idx_v,
    buf0,
    buf1,
    buf2,
    gs0,
    gs1,
    gs2,
    ws0,
    ws1,
    ws2,
):
    info = plsc.get_sparse_core_info()
    nc = info.num_cores
    wid = lax.axis_index("s") * nc + lax.axis_index("c")
    crow0 = wid * chunks_per_w
    pltpu.sync_copy(idx_hbm.at[pl.ds(crow0, chunks_per_w)], idx_v)

    bufs = (buf0, buf1, buf2)
    gs = (gs0, gs1, gs2)
    ws = (ws0, ws1, ws2)

    def wb_dst(c):
        return out_hbm.at[pl.ds((crow0 + c) * _CHUNK, _CHUNK)]

    pltpu.async_copy(states_hbm.at[idx_v.at[0]], buf0, gs0)
    pltpu.async_copy(states_hbm.at[idx_v.at[1]], buf1, gs1)

    @pl.loop(0, chunks_per_w, step=3)
    def _(c0):
        for b in range(3):
            c = c0 + b

            @pl.when(c < chunks_per_w)
            def _():
                pltpu.make_async_copy(
                    states_hbm.at[idx_v.at[c]], bufs[b], gs[b]
                ).wait()
                pltpu.async_copy(bufs[b], wb_dst(c), ws[b])

                @pl.when(c + 2 < chunks_per_w)
                def _():
                    b2 = (b + 2) % 3

                    @pl.when(c >= 1)
                    def _():
                        pltpu.make_async_copy(bufs[b2], wb_dst(c - 1), ws[b2]).wait()

                    pltpu.async_copy(states_hbm.at[idx_v.at[c + 2]], bufs[b2], gs[b2])

    for c_last in range(chunks_per_w - 3, chunks_per_w):
        b = c_last % 3
        pltpu.make_async_copy(bufs[b], wb_dst(c_last), ws[b]).wait()


def _gather_call(states, idx2d):
    nrows = idx2d.shape[0] * idx2d.shape[1]
    chunks_per_w = nrows // (_N_WORKERS * _CHUNK)
    mesh = plsc.VectorSubcoreMesh(core_axis_name="c", subcore_axis_name="s")
    f = pl.kernel(
        functools.partial(_gather_body, chunks_per_w),
        out_type=jax.ShapeDtypeStruct((nrows, ROW_D), jnp.float32),
        mesh=mesh,
        scratch_types=[
            pltpu.VMEM((chunks_per_w, _CHUNK), jnp.int32),
            pltpu.VMEM((_CHUNK, ROW_D), jnp.float32),
            pltpu.VMEM((_CHUNK, ROW_D), jnp.float32),
            pltpu.VMEM((_CHUNK, ROW_D), jnp.float32),
            pltpu.SemaphoreType.DMA,
            pltpu.SemaphoreType.DMA,
            pltpu.SemaphoreType.DMA,
            pltpu.SemaphoreType.DMA,
            pltpu.SemaphoreType.DMA,
            pltpu.SemaphoreType.DMA,
        ],
    )
    return f(states, idx2d)


def kernel(states, prob):
    p2d = prob.reshape(N_STATES, BATCH)
    flat3, pn2d = _sample_call(p2d)
    prob_new = pn2d.reshape(-1, 1)
    new_states = _gather_call(states, flat3.reshape(-1, _CHUNK))
    return new_states, prob_new


# speedup vs baseline: 1.2371x; 1.0041x over previous
import functools

import jax
import jax.numpy as jnp
import numpy as np
from jax import lax
from jax.experimental import pallas as pl
from jax.experimental.pallas import tpu as pltpu
from jax.experimental.pallas import tpu_sc as plsc

N_STATES = 128
BATCH = 1024
ROW_D = 256
ALPHA = np.float32(0.5)
UNIF_C = np.float32((1.0 - 0.5) / 128)
TINY = np.float32(np.finfo(np.float32).tiny)

_KS0 = np.uint32(0)
_KS1 = np.uint32(42)
_KS2 = np.uint32(0 ^ 42 ^ 0x1BD11BDA)
_ROTS = ((13, 15, 26, 6), (17, 29, 16, 24))


def _rotl(x, r):
    return (x << np.uint32(r)) | (x >> np.uint32(32 - r))


def _threefry_bits(x1):
    ks = (_KS0, _KS1, _KS2)
    x0 = jnp.zeros_like(x1) + ks[0]
    x1 = x1 + ks[1]
    for i in range(5):
        for r in _ROTS[i % 2]:
            x0 = x0 + x1
            x1 = _rotl(x1, r)
            x1 = x0 ^ x1
        x0 = x0 + ks[(i + 1) % 3]
        x1 = x1 + ks[(i + 2) % 3] + np.uint32(i + 1)
    return x0 ^ x1


_BC = 128


_SB = 8


def _sample_body(p_ref, idx_ref, pn_ref, ir_ref, dt_ref, pnacc_ref):
    sblk = pl.program_id(0)

    @pl.when(sblk == 0)
    def _():
        p = p_ref[...]
        rp = ALPHA * jnp.exp(p) + UNIF_C
        ir_ref[...] = np.float32(1.0) / rp
        dt_ref[...] = p - jnp.log(rp)

    iota_k = lax.broadcasted_iota(jnp.int32, (N_STATES, _BC), 0)
    iota_b = lax.broadcasted_iota(jnp.int32, (N_STATES, _BC), 1)
    flat_rows = []
    pn_rows = []
    for si in range(_SB):
        s = sblk * _SB + si
        flat_pieces = []
        pn_pieces = []
        for bc in range(BATCH // _BC):
            ir = ir_ref[:, pl.ds(bc * _BC, _BC)]
            dt = dt_ref[:, pl.ds(bc * _BC, _BC)]
            j = (
                s * (BATCH * N_STATES) + (bc * _BC + iota_b) * N_STATES + iota_k
            ).astype(jnp.uint32)
            bits = _threefry_bits(j)
            fb = (bits >> np.uint32(9)) | np.uint32(0x3F800000)
            f = lax.bitcast_convert_type(fb, jnp.float32) - np.float32(1.0)
            u = f + TINY
            score = -jnp.log(u) * ir
            mn = jnp.min(score, axis=0, keepdims=True)
            idx = jnp.min(
                jnp.where(score == mn, iota_k, np.int32(N_STATES)),
                axis=0,
                keepdims=True,
            )
            pn = jnp.sum(
                jnp.where(iota_k == idx, dt, np.float32(0.0)), axis=0, keepdims=True
            )
            flat_pieces.append(idx * BATCH + (bc * _BC + iota_b[0:1, :]))
            pn_pieces.append(pn)
        flat_rows.append(jnp.concatenate(flat_pieces, axis=1))
        pn_rows.append(jnp.concatenate(pn_pieces, axis=1))
    idx_ref[...] = jnp.concatenate(flat_rows, axis=0)
    pnacc_ref[pl.ds(sblk * _SB, _SB), :] = jnp.concatenate(pn_rows, axis=0)

    @pl.when(sblk == (N_STATES // _SB) - 1)
    def _():
        x = pnacc_ref[...]
        m = jnp.max(x, axis=0, keepdims=True)
        lse = jnp.log(jnp.sum(jnp.exp(x - m), axis=0, keepdims=True)) + m
        pn_ref[...] = x - lse


def _sample_call(p2d):
    return pl.pallas_call(
        _sample_body,
        grid=(N_STATES // _SB,),
        in_specs=[pl.BlockSpec((N_STATES, BATCH), lambda s: (0, 0))],
        out_specs=[
            pl.BlockSpec((_SB, BATCH), lambda s: (s, 0)),
            pl.BlockSpec((N_STATES, BATCH), lambda s: (0, 0)),
        ],
        out_shape=[
            jax.ShapeDtypeStruct((N_STATES, BATCH), jnp.int32),
            jax.ShapeDtypeStruct((N_STATES, BATCH), jnp.float32),
        ],
        scratch_shapes=[
            pltpu.VMEM((N_STATES, BATCH), jnp.float32),
            pltpu.VMEM((N_STATES, BATCH), jnp.float32),
            pltpu.VMEM((N_STATES, BATCH), jnp.float32),
        ],
    )(p2d)


_N_WORKERS = 32
_CHUNK = 128


def _gather_body(
    chunks_per_w,
    states_hbm,
    idx_hbm,
    out_hbm,
    idx_v,
    buf0,
    buf1,
    buf2,
    gs0,
    gs1,
    gs2,
    ws0,
    ws1,
    ws2,
):
    info = plsc.get_sparse_core_info()
    nc = info.num_cores
    wid = lax.axis_index("s") * nc + lax.axis_index("c")
    crow0 = wid * chunks_per_w
    pltpu.sync_copy(idx_hbm.at[pl.ds(crow0, chunks_per_w)], idx_v)

    bufs = (buf0, buf1, buf2)
    gs = (gs0, gs1, gs2)
    ws = (ws0, ws1, ws2)

    def wb_dst(c):
        return out_hbm.at[pl.ds((crow0 + c) * _CHUNK, _CHUNK)]

    pltpu.async_copy(states_hbm.at[idx_v.at[0]], buf0, gs0)
    pltpu.async_copy(states_hbm.at[idx_v.at[1]], buf1, gs1)

    @pl.loop(0, chunks_per_w, step=3)
    def _(c0):
        for b in range(3):
            c = c0 + b

            @pl.when(c < chunks_per_w)
            def _():
                pltpu.make_async_copy(
                    states_hbm.at[idx_v.at[c]], bufs[b], gs[b]
                ).wait()
                pltpu.async_copy(bufs[b], wb_dst(c), ws[b])

                @pl.when(c + 2 < chunks_per_w)
                def _():
                    b2 = (b + 2) % 3

                    @pl.when(c >= 1)
                    def _():
                        pltpu.make_async_copy(bufs[b2], wb_dst(c - 1), ws[b2]).wait()

                    pltpu.async_copy(states_hbm.at[idx_v.at[c + 2]], bufs[b2], gs[b2])

    for c_last in range(chunks_per_w - 3, chunks_per_w):
        b = c_last % 3
        pltpu.make_async_copy(bufs[b], wb_dst(c_last), ws[b]).wait()


def _gather_call(states, idx2d):
    nrows = idx2d.shape[0] * idx2d.shape[1]
    chunks_per_w = nrows // (_N_WORKERS * _CHUNK)
    mesh = plsc.VectorSubcoreMesh(core_axis_name="c", subcore_axis_name="s")
    f = pl.kernel(
        functools.partial(_gather_body, chunks_per_w),
        out_type=jax.ShapeDtypeStruct((nrows, ROW_D), jnp.float32),
        mesh=mesh,
        scratch_types=[
            pltpu.VMEM((chunks_per_w, _CHUNK), jnp.int32),
            pltpu.VMEM((_CHUNK, ROW_D), jnp.float32),
            pltpu.VMEM((_CHUNK, ROW_D), jnp.float32),
            pltpu.VMEM((_CHUNK, ROW_D), jnp.float32),
            pltpu.SemaphoreType.DMA,
            pltpu.SemaphoreType.DMA,
            pltpu.SemaphoreType.DMA,
            pltpu.SemaphoreType.DMA,
            pltpu.SemaphoreType.DMA,
            pltpu.SemaphoreType.DMA,
        ],
    )
    return f(states, idx2d)


def kernel(states, prob):
    p2d = prob.reshape(N_STATES, BATCH)
    flat3, pn2d = _sample_call(p2d)
    prob_new = pn2d.reshape(-1, 1)
    new_states = _gather_call(states, flat3.reshape(-1, _CHUNK))
    return new_states, prob_new


# speedup vs baseline: 1.2376x; 1.0004x over previous
import functools

import jax
import jax.numpy as jnp
import numpy as np
from jax import lax
from jax.experimental import pallas as pl
from jax.experimental.pallas import tpu as pltpu
from jax.experimental.pallas import tpu_sc as plsc

N_STATES = 128
BATCH = 1024
ROW_D = 256
ALPHA = np.float32(0.5)
UNIF_C = np.float32((1.0 - 0.5) / 128)
TINY = np.float32(np.finfo(np.float32).tiny)

_KS0 = np.uint32(0)
_KS1 = np.uint32(42)
_KS2 = np.uint32(0 ^ 42 ^ 0x1BD11BDA)
_ROTS = ((13, 15, 26, 6), (17, 29, 16, 24))


def _rotl(x, r):
    return (x << np.uint32(r)) | (x >> np.uint32(32 - r))


def _threefry_bits(x1):
    ks = (_KS0, _KS1, _KS2)
    x0 = jnp.zeros_like(x1) + ks[0]
    x1 = x1 + ks[1]
    for i in range(5):
        for r in _ROTS[i % 2]:
            x0 = x0 + x1
            x1 = _rotl(x1, r)
            x1 = x0 ^ x1
        x0 = x0 + ks[(i + 1) % 3]
        x1 = x1 + ks[(i + 2) % 3] + np.uint32(i + 1)
    return x0 ^ x1


_BC = 128


_SB = 8


def _sample_body(p_ref, idx_ref, pn_ref, ir_ref, dt_ref, pnacc_ref):
    sblk = pl.program_id(0)

    @pl.when(sblk == 0)
    def _():
        p = p_ref[...]
        rp = ALPHA * jnp.exp(p) + UNIF_C
        ir_ref[...] = np.float32(1.0) / rp
        dt_ref[...] = p - jnp.log(rp)

    iota_k = lax.broadcasted_iota(jnp.int32, (N_STATES, _BC), 0)
    iota_b = lax.broadcasted_iota(jnp.int32, (N_STATES, _BC), 1)
    jbase = (iota_b * N_STATES + iota_k).astype(jnp.uint32)
    flat_rows = []
    pn_rows = []
    for si in range(_SB):
        s = sblk * _SB + si
        flat_pieces = []
        pn_pieces = []
        for bc in range(BATCH // _BC):
            ir = ir_ref[:, pl.ds(bc * _BC, _BC)]
            dt = dt_ref[:, pl.ds(bc * _BC, _BC)]
            j = jbase + (s * (BATCH * N_STATES) + bc * _BC * N_STATES).astype(
                jnp.uint32
            )
            bits = _threefry_bits(j)
            fb = (bits >> np.uint32(9)) | np.uint32(0x3F800000)
            f = lax.bitcast_convert_type(fb, jnp.float32) - np.float32(1.0)
            u = f + TINY
            score = -jnp.log(u) * ir
            mn = jnp.min(score, axis=0, keepdims=True)
            idx = jnp.min(
                jnp.where(score == mn, iota_k, np.int32(N_STATES)),
                axis=0,
                keepdims=True,
            )
            pn = jnp.sum(
                jnp.where(iota_k == idx, dt, np.float32(0.0)), axis=0, keepdims=True
            )
            flat_pieces.append(idx * BATCH + (bc * _BC + iota_b[0:1, :]))
            pn_pieces.append(pn)
        flat_rows.append(jnp.concatenate(flat_pieces, axis=1))
        pn_rows.append(jnp.concatenate(pn_pieces, axis=1))
    idx_ref[...] = jnp.concatenate(flat_rows, axis=0)
    pnacc_ref[pl.ds(sblk * _SB, _SB), :] = jnp.concatenate(pn_rows, axis=0)

    @pl.when(sblk == (N_STATES // _SB) - 1)
    def _():
        x = pnacc_ref[...]
        m = jnp.max(x, axis=0, keepdims=True)
        lse = jnp.log(jnp.sum(jnp.exp(x - m), axis=0, keepdims=True)) + m
        pn_ref[...] = x - lse


def _sample_call(p2d):
    return pl.pallas_call(
        _sample_body,
        grid=(N_STATES // _SB,),
        in_specs=[pl.BlockSpec((N_STATES, BATCH), lambda s: (0, 0))],
        out_specs=[
            pl.BlockSpec((_SB, BATCH), lambda s: (s, 0)),
            pl.BlockSpec((N_STATES, BATCH), lambda s: (0, 0)),
        ],
        out_shape=[
            jax.ShapeDtypeStruct((N_STATES, BATCH), jnp.int32),
            jax.ShapeDtypeStruct((N_STATES, BATCH), jnp.float32),
        ],
        scratch_shapes=[
            pltpu.VMEM((N_STATES, BATCH), jnp.float32),
            pltpu.VMEM((N_STATES, BATCH), jnp.float32),
            pltpu.VMEM((N_STATES, BATCH), jnp.float32),
        ],
    )(p2d)


_N_WORKERS = 32
_CHUNK = 128


def _gather_body(
    chunks_per_w,
    states_hbm,
    idx_hbm,
    out_hbm,
    idx_v,
    buf0,
    buf1,
    buf2,
    gs0,
    gs1,
    gs2,
    ws0,
    ws1,
    ws2,
):
    info = plsc.get_sparse_core_info()
    nc = info.num_cores
    wid = lax.axis_index("s") * nc + lax.axis_index("c")
    crow0 = wid * chunks_per_w
    pltpu.sync_copy(idx_hbm.at[pl.ds(crow0, chunks_per_w)], idx_v)

    bufs = (buf0, buf1, buf2)
    gs = (gs0, gs1, gs2)
    ws = (ws0, ws1, ws2)

    def wb_dst(c):
        return out_hbm.at[pl.ds((crow0 + c) * _CHUNK, _CHUNK)]

    pltpu.async_copy(states_hbm.at[idx_v.at[0]], buf0, gs0)
    pltpu.async_copy(states_hbm.at[idx_v.at[1]], buf1, gs1)

    @pl.loop(0, chunks_per_w, step=3)
    def _(c0):
        for b in range(3):
            c = c0 + b

            @pl.when(c < chunks_per_w)
            def _():
                pltpu.make_async_copy(
                    states_hbm.at[idx_v.at[c]], bufs[b], gs[b]
                ).wait()
                pltpu.async_copy(bufs[b], wb_dst(c), ws[b])

                @pl.when(c + 2 < chunks_per_w)
                def _():
                    b2 = (b + 2) % 3

                    @pl.when(c >= 1)
                    def _():
                        pltpu.make_async_copy(bufs[b2], wb_dst(c - 1), ws[b2]).wait()

                    pltpu.async_copy(states_hbm.at[idx_v.at[c + 2]], bufs[b2], gs[b2])

    for c_last in range(chunks_per_w - 3, chunks_per_w):
        b = c_last % 3
        pltpu.make_async_copy(bufs[b], wb_dst(c_last), ws[b]).wait()


def _gather_call(states, idx2d):
    nrows = idx2d.shape[0] * idx2d.shape[1]
    chunks_per_w = nrows // (_N_WORKERS * _CHUNK)
    mesh = plsc.VectorSubcoreMesh(core_axis_name="c", subcore_axis_name="s")
    f = pl.kernel(
        functools.partial(_gather_body, chunks_per_w),
        out_type=jax.ShapeDtypeStruct((nrows, ROW_D), jnp.float32),
        mesh=mesh,
        scratch_types=[
            pltpu.VMEM((chunks_per_w, _CHUNK), jnp.int32),
            pltpu.VMEM((_CHUNK, ROW_D), jnp.float32),
            pltpu.VMEM((_CHUNK, ROW_D), jnp.float32),
            pltpu.VMEM((_CHUNK, ROW_D), jnp.float32),
            pltpu.SemaphoreType.DMA,
            pltpu.SemaphoreType.DMA,
            pltpu.SemaphoreType.DMA,
            pltpu.SemaphoreType.DMA,
            pltpu.SemaphoreType.DMA,
            pltpu.SemaphoreType.DMA,
        ],
    )
    return f(states, idx2d)


def kernel(states, prob):
    p2d = prob.reshape(N_STATES, BATCH)
    flat3, pn2d = _sample_call(p2d)
    prob_new = pn2d.reshape(-1, 1)
    new_states = _gather_call(states, flat3.reshape(-1, _CHUNK))
    return new_states, prob_new
